# Initial kernel scaffold; baseline (speedup 1.0000x reference)
import jax
import jax.numpy as jnp
from jax.experimental import pallas as pl


def kernel(atomic_numbers, gate_weight):
    raise NotImplementedError("write your pallas kernel here")


# speedup vs baseline: 4.3071x; 4.3071x over previous
import functools

import jax
import jax.numpy as jnp
import numpy as np
from jax import lax
from jax.experimental import pallas as pl
from jax.experimental.pallas import tpu as pltpu
from jax.experimental.pallas import tpu_sc as plsc

_LANES = 16


@functools.partial(jax.jit, static_argnames=("per_w", "chunk", "nelems"))
def _sc_gather(an_flat, gate_flat, *, per_w, chunk, nelems):
    total = an_flat.shape[0]
    n_chunks = per_w // chunk
    groups = chunk // _LANES

    mesh = plsc.VectorSubcoreMesh(core_axis_name="c", subcore_axis_name="s")

    @functools.partial(
        pl.kernel,
        out_type=jax.ShapeDtypeStruct((total * nelems,), jnp.float32),
        mesh=mesh,
        compiler_params=pltpu.CompilerParams(needs_layout_passes=False),
        scratch_types=[
            pltpu.VMEM((chunk,), jnp.int32),
            pltpu.VMEM((chunk * nelems,), jnp.float32),
            pltpu.VMEM((gate_flat.shape[0],), jnp.float32),
        ],
    )
    def body(an_hbm, gate_hbm, out_hbm, an_v, out_v, gate_v):
        wid = lax.axis_index("s") * 2 + lax.axis_index("c")
        base = wid * per_w
        pltpu.sync_copy(gate_hbm, gate_v)
        lane = lax.iota(jnp.int32, _LANES)
        a_consts = []
        k_consts = []
        for v in range(nelems):
            q = lane + v * _LANES
            a_v = (q * 9363) >> 16
            a_consts.append(a_v)
            k_consts.append(q - a_v * nelems)

        for c in range(n_chunks):
            pltpu.sync_copy(an_hbm.at[pl.ds(base + c * chunk, chunk)], an_v)

            def group(t, _):
                for v in range(nelems):
                    a_idx = a_consts[v] + t * _LANES
                    an_g = plsc.load_gather(an_v, [a_idx])
                    g_idx = an_g * nelems + k_consts[v]
                    val = plsc.load_gather(gate_v, [g_idx])
                    out_v[pl.ds(t * (_LANES * nelems) + v * _LANES, _LANES)] = val
                return 0

            lax.fori_loop(0, groups, group, 0)
            pltpu.sync_copy(
                out_v,
                out_hbm.at[pl.ds((base + c * chunk) * nelems, chunk * nelems)],
            )

    return body(an_flat, gate_flat)


def kernel(atomic_numbers, gate_weight):
    b, a = atomic_numbers.shape
    nelems = gate_weight.shape[1]
    total = b * a
    n_workers = 32
    per_w = total // n_workers
    chunk = 4096
    an_flat = atomic_numbers.reshape(total).astype(jnp.int32)
    gate_flat = gate_weight.reshape(-1).astype(jnp.float32)
    out_flat = _sc_gather(
        an_flat, gate_flat, per_w=per_w, chunk=chunk, nelems=nelems
    )
    return out_flat.reshape(b, a, nelems)


# speedup vs baseline: 5.1846x; 1.2037x over previous
import functools

import jax
import jax.numpy as jnp
from jax import lax
from jax.experimental import pallas as pl
from jax.experimental.pallas import tpu as pltpu
from jax.experimental.pallas import tpu_sc as plsc

_LANES = 16
_N_WORKERS = 32
_CHUNK = 4096


@functools.partial(jax.jit, static_argnames=("nelems",))
def _sc_gather(an_flat, gate_flat, *, nelems):
    total = an_flat.shape[0]
    per_w = total // _N_WORKERS
    n_chunks = per_w // _CHUNK
    groups = _CHUNK // _LANES
    group_out = _LANES * nelems
    table_n = gate_flat.shape[0]
    gather_len = table_n - nelems + 1

    mesh = plsc.VectorSubcoreMesh(core_axis_name="c", subcore_axis_name="s")

    @functools.partial(
        pl.kernel,
        out_type=jax.ShapeDtypeStruct((total * nelems,), jnp.float32),
        mesh=mesh,
        compiler_params=pltpu.CompilerParams(needs_layout_passes=False),
        scratch_types=[
            pltpu.VMEM((_CHUNK,), jnp.int32),
            pltpu.VMEM((_CHUNK,), jnp.int32),
            pltpu.VMEM((_CHUNK * nelems,), jnp.float32),
            pltpu.VMEM((_CHUNK * nelems,), jnp.float32),
            pltpu.VMEM((table_n,), jnp.float32),
            pltpu.SemaphoreType.DMA,
            pltpu.SemaphoreType.DMA,
            pltpu.SemaphoreType.DMA,
            pltpu.SemaphoreType.DMA,
        ],
    )
    def body(
        an_hbm,
        gate_hbm,
        out_hbm,
        an_v0,
        an_v1,
        out_v0,
        out_v1,
        gate_v,
        sem_i0,
        sem_i1,
        sem_o0,
        sem_o1,
    ):
        wid = lax.axis_index("s") * 2 + lax.axis_index("c")
        base = wid * per_w
        pltpu.sync_copy(gate_hbm, gate_v)

        an_bufs = (an_v0, an_v1)
        out_bufs = (out_v0, out_v1)
        sems_i = (sem_i0, sem_i1)
        sems_o = (sem_o0, sem_o1)

        lane7 = lax.iota(jnp.int32, _LANES) * nelems
        pos = [lane7 + k for k in range(nelems)]

        def start_in(ci, p):
            return pltpu.async_copy(
                an_hbm.at[pl.ds(base + ci * _CHUNK, _CHUNK)], an_bufs[p], sems_i[p]
            )

        h_in = [start_in(0, 0), None]
        h_out = [None, None]

        for ci in range(n_chunks):
            p = ci % 2
            if ci + 1 < n_chunks:
                h_in[1 - p] = start_in(ci + 1, 1 - p)
            h_in[p].wait()
            if h_out[p] is not None:
                h_out[p].wait()
            an_v = an_bufs[p]
            out_v = out_bufs[p]

            @plsc.parallel_loop(0, groups, 1, unroll=4)
            def group(t):
                an7 = an_v[pl.ds(t * _LANES, _LANES)] * nelems
                dst = out_v.at[pl.ds(t * group_out, group_out)]
                for k in range(nelems):
                    val = plsc.load_gather(gate_v, [an7 + k])
                    plsc.store_scatter(dst, [pos[k]], val)

            h_out[p] = pltpu.async_copy(
                out_v,
                out_hbm.at[pl.ds((base + ci * _CHUNK) * nelems, _CHUNK * nelems)],
                sems_o[p],
            )

        for h in h_out:
            if h is not None:
                h.wait()

    return body(an_flat, gate_flat)


def kernel(atomic_numbers, gate_weight):
    b, a = atomic_numbers.shape
    nelems = gate_weight.shape[1]
    total = b * a
    an_flat = atomic_numbers.reshape(total).astype(jnp.int32)
    gate_flat = gate_weight.reshape(-1).astype(jnp.float32)
    out_flat = _sc_gather(an_flat, gate_flat, nelems=nelems)
    return out_flat.reshape(b, a, nelems)


# speedup vs baseline: 188.6656x; 36.3898x over previous
import functools

import jax
import jax.numpy as jnp
from jax import lax
from jax.experimental import pallas as pl
from jax.experimental.pallas import tpu as pltpu
from jax.experimental.pallas import tpu_sc as plsc

_LANES = 16
_N_WORKERS = 32
_CHUNK = 4096


@functools.partial(jax.jit, static_argnames=("nelems",))
def _sc_gather(an_phys, gate_cols, *, nelems):
    total = an_phys.shape[0]
    table_n = gate_cols.shape[0] // nelems
    per_w = total // _N_WORKERS
    n_chunks = per_w // _CHUNK
    groups = _CHUNK // _LANES

    mesh = plsc.VectorSubcoreMesh(core_axis_name="c", subcore_axis_name="s")

    @functools.partial(
        pl.kernel,
        out_type=jax.ShapeDtypeStruct((nelems * total,), jnp.float32),
        mesh=mesh,
        compiler_params=pltpu.CompilerParams(needs_layout_passes=False),
        scratch_types=[
            pltpu.VMEM((_CHUNK,), jnp.int32),
            pltpu.VMEM((_CHUNK,), jnp.int32),
            pltpu.VMEM((nelems * _CHUNK,), jnp.float32),
            pltpu.VMEM((nelems * _CHUNK,), jnp.float32),
            pltpu.VMEM((nelems * table_n,), jnp.float32),
            pltpu.SemaphoreType.DMA,
            pltpu.SemaphoreType.DMA,
            pltpu.SemaphoreType.DMA,
            pltpu.SemaphoreType.DMA,
        ],
    )
    def body(
        an_hbm,
        gate_hbm,
        out_hbm,
        an_v0,
        an_v1,
        out_v0,
        out_v1,
        cols_v,
        sem_i0,
        sem_i1,
        sem_o0,
        sem_o1,
    ):
        wid = lax.axis_index("s") * 2 + lax.axis_index("c")
        base = wid * per_w
        pltpu.sync_copy(gate_hbm, cols_v)

        an_bufs = (an_v0, an_v1)
        out_bufs = (out_v0, out_v1)
        sems_i = (sem_i0, sem_i1)
        sems_o = (sem_o0, sem_o1)

        def start_in(ci, p):
            return pltpu.async_copy(
                an_hbm.at[pl.ds(base + ci * _CHUNK, _CHUNK)],
                an_bufs[p],
                sems_i[p],
            )

        h_in = [start_in(0, 0), None]
        h_out = [None, None]

        for ci in range(n_chunks):
            p = ci % 2
            if ci + 1 < n_chunks:
                h_in[1 - p] = start_in(ci + 1, 1 - p)
            h_in[p].wait()
            if h_out[p] is not None:
                for h in h_out[p]:
                    h.wait()
            an_v = an_bufs[p]
            out_v = out_bufs[p]

            @plsc.parallel_loop(0, groups, 1, unroll=4)
            def group(t):
                an16 = an_v[pl.ds(t * _LANES, _LANES)]
                for k in range(nelems):
                    val = plsc.load_gather(cols_v, [an16 + k * table_n])
                    out_v[pl.ds(k * _CHUNK + t * _LANES, _LANES)] = val

            h_out[p] = [
                pltpu.async_copy(
                    out_v.at[pl.ds(k * _CHUNK, _CHUNK)],
                    out_hbm.at[pl.ds(k * total + base + ci * _CHUNK, _CHUNK)],
                    sems_o[p],
                )
                for k in range(nelems)
            ]

        for hs in h_out:
            if hs is not None:
                for h in hs:
                    h.wait()

    return body(an_phys, gate_cols)


def kernel(atomic_numbers, gate_weight):
    b, a = atomic_numbers.shape
    nelems = gate_weight.shape[1]
    total = b * a
    tb, bc = b // 128, 128
    ta, ar = a // 8, 8
    an_phys = (
        atomic_numbers.astype(jnp.int32)
        .reshape(tb, bc, ta, ar)
        .transpose(2, 0, 3, 1)
        .reshape(total)
    )
    gate_cols = gate_weight.astype(jnp.float32).T.reshape(-1)
    out_planes = _sc_gather(an_phys, gate_cols, nelems=nelems)
    return (
        out_planes.reshape(nelems, ta, tb, ar, bc)
        .transpose(2, 4, 1, 3, 0)
        .reshape(b, a, nelems)
    )


# speedup vs baseline: 195.1964x; 1.0346x over previous
import functools

import jax
import jax.numpy as jnp
from jax import lax
from jax.experimental import pallas as pl
from jax.experimental.pallas import tpu as pltpu
from jax.experimental.pallas import tpu_sc as plsc

_LANES = 16
_N_WORKERS = 32
_CHUNK = 6400
_COL_STRIDE = 24


@functools.partial(jax.jit, static_argnames=("nelems",))
def _sc_gather(an_phys, gate_cols, *, nelems):
    total = an_phys.shape[0]
    table_n = gate_cols.shape[0] // nelems
    per_w = total // _N_WORKERS
    n_chunks = per_w // _CHUNK
    groups = _CHUNK // _LANES

    mesh = plsc.VectorSubcoreMesh(core_axis_name="c", subcore_axis_name="s")

    @functools.partial(
        pl.kernel,
        out_type=jax.ShapeDtypeStruct((nelems * total,), jnp.float32),
        mesh=mesh,
        compiler_params=pltpu.CompilerParams(needs_layout_passes=False),
        scratch_types=[
            pltpu.VMEM((_CHUNK,), jnp.int32),
            pltpu.VMEM((_CHUNK,), jnp.int32),
            pltpu.VMEM((nelems * _CHUNK,), jnp.float32),
            pltpu.VMEM((nelems * _CHUNK,), jnp.float32),
            pltpu.VMEM((nelems * table_n,), jnp.float32),
            pltpu.SemaphoreType.DMA,
            pltpu.SemaphoreType.DMA,
            pltpu.SemaphoreType.DMA,
            pltpu.SemaphoreType.DMA,
        ],
    )
    def body(
        an_hbm,
        gate_hbm,
        out_hbm,
        an_v0,
        an_v1,
        out_v0,
        out_v1,
        cols_v,
        sem_i0,
        sem_i1,
        sem_o0,
        sem_o1,
    ):
        wid = lax.axis_index("s") * 2 + lax.axis_index("c")
        base = wid * per_w
        pltpu.sync_copy(gate_hbm, cols_v)

        an_bufs = (an_v0, an_v1)
        out_bufs = (out_v0, out_v1)
        sems_i = (sem_i0, sem_i1)
        sems_o = (sem_o0, sem_o1)

        def start_in(ci, p):
            return pltpu.async_copy(
                an_hbm.at[pl.ds(base + ci * _CHUNK, _CHUNK)],
                an_bufs[p],
                sems_i[p],
            )

        h_in = [start_in(0, 0), None]
        h_out = [None, None]

        for ci in range(n_chunks):
            p = ci % 2
            if ci + 1 < n_chunks:
                h_in[1 - p] = start_in(ci + 1, 1 - p)
            h_in[p].wait()
            if h_out[p] is not None:
                for h in h_out[p]:
                    h.wait()
            an_v = an_bufs[p]
            out_v = out_bufs[p]

            @plsc.parallel_loop(0, groups, 1, unroll=8)
            def group(t):
                an16 = an_v[pl.ds(t * _LANES, _LANES)]
                for k in range(nelems):
                    val = plsc.load_gather(
                        cols_v.at[pl.ds(k * table_n, _LANES + 2)], [an16]
                    )
                    out_v[pl.ds(k * _CHUNK + t * _LANES, _LANES)] = val

            h_out[p] = [
                pltpu.async_copy(
                    out_v.at[pl.ds(k * _CHUNK, _CHUNK)],
                    out_hbm.at[pl.ds(k * total + base + ci * _CHUNK, _CHUNK)],
                    sems_o[p],
                )
                for k in range(nelems)
            ]

        for hs in h_out:
            if hs is not None:
                for h in hs:
                    h.wait()

    return body(an_phys, gate_cols)


def kernel(atomic_numbers, gate_weight):
    b, a = atomic_numbers.shape
    nelems = gate_weight.shape[1]
    total = b * a
    tb, bc = b // 128, 128
    ta, ar = a // 8, 8
    an_phys = (
        atomic_numbers.astype(jnp.int32)
        .reshape(tb, bc, ta, ar)
        .transpose(2, 0, 3, 1)
        .reshape(total)
    )
    gate_cols = jnp.pad(
        gate_weight.astype(jnp.float32).T, ((0, 0), (0, _COL_STRIDE - 18))
    ).reshape(-1)
    out_planes = _sc_gather(an_phys, gate_cols, nelems=nelems)
    return (
        out_planes.reshape(nelems, ta, tb, ar, bc)
        .transpose(2, 4, 1, 3, 0)
        .reshape(b, a, nelems)
    )
